# baseline (speedup 1.0000x reference)
import functools

import jax
import jax.numpy as jnp
from jax.experimental import pallas as pl
from jax.experimental.pallas import tpu as pltpu

_HEADS = 4
_N = 4


def _gat_kernel(x_ref, w_ref, asrc_ref, adst_ref, bias_ref, m_ref, o_ref,
                *, apply_elu):
    x = x_ref[...]
    w = w_ref[...]
    xp = jnp.dot(x, w, preferred_element_type=jnp.float32)
    C = xp.shape[1] // _HEADS

    outb = (xp[:, 0:C] + xp[:, C:2 * C] + xp[:, 2 * C:3 * C]
            + xp[:, 3 * C:4 * C]) * 0.25

    xp4 = xp[0:_N, :]
    ps = xp4 * asrc_ref[...]
    pd = xp4 * adst_ref[...]
    ones = jnp.ones((_N, 1), jnp.float32)
    m = m_ref[...]
    dn_outer = (((1,), (1,)), ((), ()))
    dn_srcsum = (((0,), (0,)), ((), ()))

    acc = jnp.zeros((_N, C), jnp.float32)
    for h in range(_HEADS):
        asr = jnp.sum(ps[:, h * C:(h + 1) * C], axis=1, keepdims=True)
        adt = jnp.sum(pd[:, h * C:(h + 1) * C], axis=1, keepdims=True)
        e = (jax.lax.dot_general(asr, ones, dn_outer)
             + jax.lax.dot_general(ones, adt, dn_outer))
        e = jnp.where(e > 0.0, e, 0.2 * e)
        e = jnp.where(m > 0.5, e, -1e30)
        emax = jnp.max(e, axis=0, keepdims=True)
        ee = jnp.exp(e - emax)
        alpha = ee / jnp.sum(ee, axis=0, keepdims=True)
        acc = acc + jax.lax.dot_general(alpha, xp4[:, h * C:(h + 1) * C],
                                        dn_srcsum)
    outn = acc * (1.0 / _HEADS)

    bias = bias_ref[...]
    out = outb + bias
    outn = outn + bias
    if apply_elu:
        out = jnp.where(out > 0.0, out, jnp.exp(out) - 1.0)
        outn = jnp.where(outn > 0.0, outn, jnp.exp(outn) - 1.0)
    o_ref[...] = out
    o_ref[0:_N, :] = outn


def _gat_layer(x, w, a_src, a_dst, bias, mask, apply_elu):
    R = x.shape[0]
    C = w.shape[1] // _HEADS
    return pl.pallas_call(
        functools.partial(_gat_kernel, apply_elu=apply_elu),
        out_shape=jax.ShapeDtypeStruct((R, C), jnp.float32),
        compiler_params=pltpu.CompilerParams(
            vmem_limit_bytes=100 * 1024 * 1024),
    )(x, w, a_src.reshape(1, -1), a_dst.reshape(1, -1),
      bias.reshape(1, -1), mask)


def _gru_heads_kernel(xgts_ref, wih0_ref, whh0_ref, bih0_ref, bhh0_ref,
                      wih1_ref, whh1_ref, bih1_ref, bhh1_ref,
                      fw1t_ref, fb1t_ref, fw2t_ref, fb2_ref, o_ref):
    wih0 = wih0_ref[...]
    whh0 = whh0_ref[...]
    bih0 = bih0_ref[...]
    bhh0 = bhh0_ref[...]
    wih1 = wih1_ref[...]
    whh1 = whh1_ref[...]
    bih1 = bih1_ref[...]
    bhh1 = bhh1_ref[...]
    B = xgts_ref.shape[1]
    H = whh0.shape[1]
    T = H
    bf = jnp.bfloat16
    whh0b = whh0.astype(bf)
    wih1b = wih1.astype(bf)
    whh1b = whh1.astype(bf)

    def gru_cell(gi, gh, h):
        r = jax.nn.sigmoid(gi[0:H] + gh[0:H])
        z = jax.nn.sigmoid(gi[H:2 * H] + gh[H:2 * H])
        n = jnp.tanh(gi[2 * H:3 * H] + r * gh[2 * H:3 * H])
        return (1.0 - z) * n + z * h

    def step(t, carry):
        h1, h2 = carry
        xt = jnp.concatenate(
            [xgts_ref[pl.ds(t + n * T, 1), :] for n in range(_N)],
            axis=0)
        gi1 = jnp.dot(wih0, xt, preferred_element_type=jnp.float32) + bih0
        gh1 = jnp.dot(whh0b, h1.astype(bf),
                      preferred_element_type=jnp.float32) + bhh0
        h1n = gru_cell(gi1, gh1, h1)
        gi2 = jnp.dot(wih1b, h1n.astype(bf),
                      preferred_element_type=jnp.float32) + bih1
        gh2 = jnp.dot(whh1b, h2.astype(bf),
                      preferred_element_type=jnp.float32) + bhh1
        h2n = gru_cell(gi2, gh2, h2)
        return (h1n, h2n)

    h0 = jnp.zeros((H, B), jnp.float32)
    _, xl = jax.lax.fori_loop(0, T, step, (h0, h0))

    fw1t = fw1t_ref[...]
    fb1t = fb1t_ref[...]
    fw2t = fw2t_ref[...]
    preds = []
    for i in range(_N):
        f = jnp.concatenate([xgts_ref[i * T:(i + 1) * T, :], xl],
                            axis=0)
        hh = jnp.dot(fw1t[i * H:(i + 1) * H, :], f,
                     preferred_element_type=jnp.float32) + fb1t[:, i:i + 1]
        hh = jnp.maximum(hh, 0.0)
        preds.append(jnp.dot(fw2t[i:i + 1, :], hh,
                             preferred_element_type=jnp.float32))
    o_ref[...] = jnp.concatenate(preds, axis=0) + fb2_ref[...]


def _bn_ncl(h, g, b):
    m = h.mean((0, 2), keepdims=True)
    v = ((h - m) ** 2).mean((0, 2), keepdims=True)
    return (h - m) / jnp.sqrt(v + 1e-5) * g[None, :, None] + b[None, :, None]


def _bn_nf(h, g, b):
    m = h.mean(0, keepdims=True)
    v = ((h - m) ** 2).mean(0, keepdims=True)
    return (h - m) / jnp.sqrt(v + 1e-5) * g[None, :] + b[None, :]


def kernel(x, y, node_feas, batch, c1w, c1b, c2w, c2b, bn1g, bn1b, bn2g, bn2b, fcw, fcb, bn3g, bn3b, fow, fob, fcw2, fcb2, g1W, g1s, g1d, g1b, g2W, g2s, g2d, g2b, wih0, whh0, bih0, bhh0, wih1, whh1, bih1, bhh1, fw1, fb1, fw2, fb2):
    n_nodes = _N
    nf = node_feas.reshape(n_nodes, 1, -1)
    h = jax.lax.conv_general_dilated(nf, c1w, (1,), 'VALID',
                                     dimension_numbers=('NCH', 'OIH', 'NCH'))
    h = jax.nn.relu(h + c1b[None, :, None])
    h = _bn_ncl(h, bn1g, bn1b)
    h = jax.lax.conv_general_dilated(h, c2w, (1,), 'VALID',
                                     dimension_numbers=('NCH', 'OIH', 'NCH'))
    h = jax.nn.relu(h + c2b[None, :, None])
    h = _bn_ncl(h, bn2g, bn2b)
    h = h.reshape(n_nodes, -1)
    h = jax.nn.relu(h @ fcw + fcb[None, :])
    h = _bn_nf(h, bn3g, bn3b)
    send = jnp.tile(h, (n_nodes, 1))
    recv = jnp.repeat(h, n_nodes, axis=0)
    h2 = jnp.concatenate([send, recv], axis=1)
    h2 = jax.nn.relu(h2 @ fow + fob[None, :])
    logits = h2 @ fcw2 + fcb2[None, :]
    U = jax.random.uniform(jax.random.key(123), logits.shape, jnp.float32)
    gnoise = -jnp.log(-jnp.log(U + 1e-10) + 1e-10)
    ysoft = jax.nn.softmax((logits + gnoise) / 0.5, axis=-1)
    kk = jnp.argmax(ysoft, axis=-1)
    yhard = jax.nn.one_hot(kk, 2, dtype=ysoft.dtype)
    ygum = jax.lax.stop_gradient(yhard - ysoft) + ysoft
    adj = ygum[:, 0].reshape(n_nodes, n_nodes)
    adj = adj * (1.0 - jnp.eye(n_nodes, dtype=adj.dtype))
    adj = jax.lax.stop_gradient(adj)
    mask = adj + jnp.eye(n_nodes, dtype=adj.dtype)

    h1 = _gat_layer(x, g1W, g1s, g1d, g1b, mask, apply_elu=True)
    hg = _gat_layer(h1, g2W, g2s, g2d, g2b, mask, apply_elu=False)

    B = batch.shape[0] // n_nodes
    xg = hg.reshape(B, n_nodes, -1)
    xgts = xg.transpose(1, 2, 0).reshape(n_nodes * 128, B)
    fw1t = fw1.transpose(0, 2, 1).reshape(n_nodes * 128, 256)
    fb1t = fb1.T
    fw2t = fw2[:, :, 0]
    predT = pl.pallas_call(
        _gru_heads_kernel,
        out_shape=jax.ShapeDtypeStruct((n_nodes, B), jnp.float32),
        compiler_params=pltpu.CompilerParams(
            vmem_limit_bytes=100 * 1024 * 1024),
    )(xgts, wih0, whh0, bih0.reshape(-1, 1), bhh0.reshape(-1, 1),
      wih1, whh1, bih1.reshape(-1, 1), bhh1.reshape(-1, 1),
      fw1t, fb1t, fw2t, fb2.reshape(n_nodes, 1))

    pred = predT.T[:, :, None]
    yr = y.reshape(B, n_nodes, -1)
    return pred, yr


# speedup vs baseline: 1.1298x; 1.1298x over previous
import functools

import jax
import jax.numpy as jnp
from jax.experimental import pallas as pl
from jax.experimental.pallas import tpu as pltpu

_HEADS = 4
_N = 4


def _gat_kernel(x_ref, w_ref, asrc_ref, adst_ref, bias_ref, m_ref, o_ref,
                *, apply_elu):
    x = x_ref[...]
    w = w_ref[...]
    xp = jnp.dot(x, w, preferred_element_type=jnp.float32)
    C = xp.shape[1] // _HEADS

    outb = (xp[:, 0:C] + xp[:, C:2 * C] + xp[:, 2 * C:3 * C]
            + xp[:, 3 * C:4 * C]) * 0.25

    xp4 = xp[0:_N, :]
    ps = xp4 * asrc_ref[...]
    pd = xp4 * adst_ref[...]
    ones = jnp.ones((_N, 1), jnp.float32)
    m = m_ref[...]
    dn_outer = (((1,), (1,)), ((), ()))
    dn_srcsum = (((0,), (0,)), ((), ()))

    acc = jnp.zeros((_N, C), jnp.float32)
    for h in range(_HEADS):
        asr = jnp.sum(ps[:, h * C:(h + 1) * C], axis=1, keepdims=True)
        adt = jnp.sum(pd[:, h * C:(h + 1) * C], axis=1, keepdims=True)
        e = (jax.lax.dot_general(asr, ones, dn_outer)
             + jax.lax.dot_general(ones, adt, dn_outer))
        e = jnp.where(e > 0.0, e, 0.2 * e)
        e = jnp.where(m > 0.5, e, -1e30)
        emax = jnp.max(e, axis=0, keepdims=True)
        ee = jnp.exp(e - emax)
        alpha = ee / jnp.sum(ee, axis=0, keepdims=True)
        acc = acc + jax.lax.dot_general(alpha, xp4[:, h * C:(h + 1) * C],
                                        dn_srcsum)
    outn = acc * (1.0 / _HEADS)

    bias = bias_ref[...]
    out = outb + bias
    outn = outn + bias
    if apply_elu:
        out = jnp.where(out > 0.0, out, jnp.exp(out) - 1.0)
        outn = jnp.where(outn > 0.0, outn, jnp.exp(outn) - 1.0)
    o_ref[...] = out
    o_ref[0:_N, :] = outn


def _gat_layer(x, w, a_src, a_dst, bias, mask, apply_elu):
    R = x.shape[0]
    C = w.shape[1] // _HEADS
    return pl.pallas_call(
        functools.partial(_gat_kernel, apply_elu=apply_elu),
        out_shape=jax.ShapeDtypeStruct((R, C), jnp.float32),
        compiler_params=pltpu.CompilerParams(
            vmem_limit_bytes=100 * 1024 * 1024),
    )(x, w, a_src.reshape(1, -1), a_dst.reshape(1, -1),
      bias.reshape(1, -1), mask)


def _gru_heads_kernel(xgts_ref, wih0_ref, whh0_ref, bih0_ref, bhh0_ref,
                      wih1_ref, whh1_ref, bih1_ref, bhh1_ref,
                      fw1t_ref, fb1t_ref, fw2t_ref, fb2_ref, o_ref):
    wih0 = wih0_ref[...]
    whh0 = whh0_ref[...]
    bih0 = bih0_ref[...]
    bhh0 = bhh0_ref[...]
    wih1 = wih1_ref[...]
    whh1 = whh1_ref[...]
    bih1 = bih1_ref[...]
    bhh1 = bhh1_ref[...]
    B = xgts_ref.shape[1]
    H = whh0.shape[1]
    T = H

    def gru_cell(gi, gh, h):
        r = jax.nn.sigmoid(gi[0:H] + gh[0:H])
        z = jax.nn.sigmoid(gi[H:2 * H] + gh[H:2 * H])
        n = jnp.tanh(gi[2 * H:3 * H] + r * gh[2 * H:3 * H])
        return (1.0 - z) * n + z * h

    def step(t, carry):
        h1, h2 = carry
        gi2 = jnp.dot(wih1, h1, preferred_element_type=jnp.float32) + bih1
        gh2 = jnp.dot(whh1, h2, preferred_element_type=jnp.float32) + bhh1
        h2n = gru_cell(gi2, gh2, h2)
        h2n = jnp.where(t > 0, h2n, h2)

        tc = jnp.minimum(t, T - 1)
        xt = jnp.concatenate(
            [xgts_ref[pl.ds(tc + n * T, 1), :] for n in range(_N)],
            axis=0)
        gi1 = jnp.dot(wih0, xt, preferred_element_type=jnp.float32) + bih0
        gh1 = jnp.dot(whh0, h1, preferred_element_type=jnp.float32) + bhh0
        h1n = gru_cell(gi1, gh1, h1)
        return (h1n, h2n)

    h0 = jnp.zeros((H, B), jnp.float32)
    _, xl = jax.lax.fori_loop(0, T + 1, step, (h0, h0))

    fw1t = fw1t_ref[...]
    fb1t = fb1t_ref[...]
    fw2t = fw2t_ref[...]
    preds = []
    for i in range(_N):
        f = jnp.concatenate([xgts_ref[i * T:(i + 1) * T, :], xl],
                            axis=0)
        hh = jnp.dot(fw1t[i * H:(i + 1) * H, :], f,
                     preferred_element_type=jnp.float32) + fb1t[:, i:i + 1]
        hh = jnp.maximum(hh, 0.0)
        preds.append(jnp.dot(fw2t[i:i + 1, :], hh,
                             preferred_element_type=jnp.float32))
    o_ref[...] = jnp.concatenate(preds, axis=0) + fb2_ref[...]


def _bn_ncl(h, g, b):
    m = h.mean((0, 2), keepdims=True)
    v = ((h - m) ** 2).mean((0, 2), keepdims=True)
    return (h - m) / jnp.sqrt(v + 1e-5) * g[None, :, None] + b[None, :, None]


def _bn_nf(h, g, b):
    m = h.mean(0, keepdims=True)
    v = ((h - m) ** 2).mean(0, keepdims=True)
    return (h - m) / jnp.sqrt(v + 1e-5) * g[None, :] + b[None, :]


def kernel(x, y, node_feas, batch, c1w, c1b, c2w, c2b, bn1g, bn1b, bn2g, bn2b, fcw, fcb, bn3g, bn3b, fow, fob, fcw2, fcb2, g1W, g1s, g1d, g1b, g2W, g2s, g2d, g2b, wih0, whh0, bih0, bhh0, wih1, whh1, bih1, bhh1, fw1, fb1, fw2, fb2):
    n_nodes = _N
    nf = node_feas.reshape(n_nodes, 1, -1)
    h = jax.lax.conv_general_dilated(nf, c1w, (1,), 'VALID',
                                     dimension_numbers=('NCH', 'OIH', 'NCH'))
    h = jax.nn.relu(h + c1b[None, :, None])
    h = _bn_ncl(h, bn1g, bn1b)
    h = jax.lax.conv_general_dilated(h, c2w, (1,), 'VALID',
                                     dimension_numbers=('NCH', 'OIH', 'NCH'))
    h = jax.nn.relu(h + c2b[None, :, None])
    h = _bn_ncl(h, bn2g, bn2b)
    h = h.reshape(n_nodes, -1)
    h = jax.nn.relu(h @ fcw + fcb[None, :])
    h = _bn_nf(h, bn3g, bn3b)
    send = jnp.tile(h, (n_nodes, 1))
    recv = jnp.repeat(h, n_nodes, axis=0)
    h2 = jnp.concatenate([send, recv], axis=1)
    h2 = jax.nn.relu(h2 @ fow + fob[None, :])
    logits = h2 @ fcw2 + fcb2[None, :]
    U = jax.random.uniform(jax.random.key(123), logits.shape, jnp.float32)
    gnoise = -jnp.log(-jnp.log(U + 1e-10) + 1e-10)
    ysoft = jax.nn.softmax((logits + gnoise) / 0.5, axis=-1)
    kk = jnp.argmax(ysoft, axis=-1)
    yhard = jax.nn.one_hot(kk, 2, dtype=ysoft.dtype)
    ygum = jax.lax.stop_gradient(yhard - ysoft) + ysoft
    adj = ygum[:, 0].reshape(n_nodes, n_nodes)
    adj = adj * (1.0 - jnp.eye(n_nodes, dtype=adj.dtype))
    adj = jax.lax.stop_gradient(adj)
    mask = adj + jnp.eye(n_nodes, dtype=adj.dtype)

    h1 = _gat_layer(x, g1W, g1s, g1d, g1b, mask, apply_elu=True)
    hg = _gat_layer(h1, g2W, g2s, g2d, g2b, mask, apply_elu=False)

    B = batch.shape[0] // n_nodes
    xg = hg.reshape(B, n_nodes, -1)
    xgts = xg.transpose(1, 2, 0).reshape(n_nodes * 128, B)
    fw1t = fw1.transpose(0, 2, 1).reshape(n_nodes * 128, 256)
    fb1t = fb1.T
    fw2t = fw2[:, :, 0]
    predT = pl.pallas_call(
        _gru_heads_kernel,
        out_shape=jax.ShapeDtypeStruct((n_nodes, B), jnp.float32),
        compiler_params=pltpu.CompilerParams(
            vmem_limit_bytes=100 * 1024 * 1024),
    )(xgts, wih0, whh0, bih0.reshape(-1, 1), bhh0.reshape(-1, 1),
      wih1, whh1, bih1.reshape(-1, 1), bhh1.reshape(-1, 1),
      fw1t, fb1t, fw2t, fb2.reshape(n_nodes, 1))

    pred = predT.T[:, :, None]
    yr = y.reshape(B, n_nodes, -1)
    return pred, yr
